# baseline (speedup 1.0000x reference)
import jax
import jax.numpy as jnp
from jax import lax
from jax.experimental import pallas as pl
from jax.experimental.pallas import tpu as pltpu
from jax.experimental.pallas import tpu_sc as plsc

N = 10000
E = 320000
D = 128
H = 128
L = 3
EPS = 1e-30
ALPHA = 0.5

NC = 2
NS = 16
NW = NC * NS
NOUT = 10112
CHUNK = 128
EPT = E // NW
NFC = EPT // CHUNK
REM = EPT - NFC * CHUNK
ZROWS = NOUT // NS
ZR = 32

_mesh = plsc.VectorSubcoreMesh(core_axis_name="c", subcore_axis_name="s")


def _segsum_body(eh_hbm, src_hbm, dst_hbm, z_hbm, out_hbm,
                 table, sidx3, dstA, dstB, dstC, rowsA, rowsB, rowsC,
                 dstt, gsA, gsB, gsC, ssA, ssB, ssC, sxA, sxB, sxC):
    cid = lax.axis_index("c")
    sid = lax.axis_index("s")
    wid = sid * NC + cid
    ebase = wid * EPT
    zbase = sid * ZROWS

    pltpu.sync_copy(z_hbm, table.at[pl.ds(zbase, ZROWS)])

    plsc.subcore_barrier()

    def _triple(i, _):
        off = ebase + (i * 3) * CHUNK
        pltpu.sync_copy(src_hbm.at[pl.ds(off, 3 * CHUNK)], sidx3)
        dA = pltpu.async_copy(eh_hbm.at[sidx3.at[pl.ds(0, CHUNK)]],
                              rowsA, gsA)
        dB = pltpu.async_copy(eh_hbm.at[sidx3.at[pl.ds(CHUNK, CHUNK)]],
                              rowsB, gsB)
        dC = pltpu.async_copy(eh_hbm.at[sidx3.at[pl.ds(2 * CHUNK, CHUNK)]],
                              rowsC, gsC)
        xA = pltpu.async_copy(dst_hbm.at[pl.ds(off, CHUNK)], dstA, sxA)
        xB = pltpu.async_copy(dst_hbm.at[pl.ds(off + CHUNK, CHUNK)], dstB,
                              sxB)
        xC = pltpu.async_copy(dst_hbm.at[pl.ds(off + 2 * CHUNK, CHUNK)],
                              dstC, sxC)
        dA.wait()
        xA.wait()
        sA = pltpu.async_copy(rowsA, table.at[dstA], ssA, add=True)
        dB.wait()
        xB.wait()
        sB = pltpu.async_copy(rowsB, table.at[dstB], ssB, add=True)
        dC.wait()
        xC.wait()
        sC = pltpu.async_copy(rowsC, table.at[dstC], ssC, add=True)
        sA.wait()
        sB.wait()
        sC.wait()
        return 0

    lax.fori_loop(0, NFC // 3, _triple, 0)

    toff = ebase + NFC * CHUNK
    pltpu.sync_copy(src_hbm.at[pl.ds(toff, REM)], sidx3.at[pl.ds(0, REM)])
    pltpu.sync_copy(dst_hbm.at[pl.ds(toff, REM)], dstt)
    pltpu.async_copy(eh_hbm.at[sidx3.at[pl.ds(0, REM)]],
                     rowsA.at[pl.ds(0, REM)], gsA).wait()
    pltpu.sync_copy(rowsA.at[pl.ds(0, REM)], table.at[dstt], add=True)

    plsc.subcore_barrier()

    pltpu.sync_copy(table.at[pl.ds(zbase, ZROWS)],
                    out_hbm.at[cid, pl.ds(zbase, ZROWS)])


_segsum = pl.kernel(
    _segsum_body,
    out_type=jax.ShapeDtypeStruct((NC, NOUT, H), jnp.float32),
    mesh=_mesh,
    scratch_types=[
        pltpu.VMEM_SHARED((NOUT, H), jnp.float32),
        pltpu.VMEM((3 * CHUNK,), jnp.int32),
        pltpu.VMEM((CHUNK,), jnp.int32),
        pltpu.VMEM((CHUNK,), jnp.int32),
        pltpu.VMEM((CHUNK,), jnp.int32),
        pltpu.VMEM((CHUNK, H), jnp.float32),
        pltpu.VMEM((CHUNK, H), jnp.float32),
        pltpu.VMEM((CHUNK, H), jnp.float32),
        pltpu.VMEM((REM,), jnp.int32),
        pltpu.SemaphoreType.DMA,
        pltpu.SemaphoreType.DMA,
        pltpu.SemaphoreType.DMA,
        pltpu.SemaphoreType.DMA,
        pltpu.SemaphoreType.DMA,
        pltpu.SemaphoreType.DMA,
        pltpu.SemaphoreType.DMA,
        pltpu.SemaphoreType.DMA,
        pltpu.SemaphoreType.DMA,
    ],
)


RB = 1000
_GRID = N // RB


def _proj_body(x_ref, w_ref, b_ref, h_ref, eh_ref):
    h = lax.dot_general(x_ref[...], w_ref[...], (((1,), (0,)), ((), ())),
                        precision=lax.Precision.HIGHEST,
                        preferred_element_type=jnp.float32) + b_ref[...]
    h_ref[...] = h
    eh_ref[...] = jnp.exp(h)


def _dense_body(h_ref, s0_ref, s1_ref, wt_ref, wb_ref, b_ref, g_ref, be_ref,
                hout_ref, ehout_ref):
    h = h_ref[...]
    s = s0_ref[...] + s1_ref[...]
    agg = jnp.where(s > 0, jnp.log(jnp.maximum(s, EPS)), 0.0)
    hn = (lax.dot_general(h, wt_ref[...], (((1,), (0,)), ((), ())),
                          precision=lax.Precision.HIGHEST,
                          preferred_element_type=jnp.float32)
          + lax.dot_general(agg, wb_ref[...], (((1,), (0,)), ((), ())),
                            precision=lax.Precision.HIGHEST,
                            preferred_element_type=jnp.float32)
          + b_ref[...])
    mu = jnp.mean(hn, axis=1, keepdims=True)
    var = jnp.mean((hn - mu) ** 2, axis=1, keepdims=True)
    hn = (hn - mu) / jnp.sqrt(var + 1e-5) * g_ref[...] + be_ref[...]
    hn = jnp.maximum(hn, 0.0)
    hnew = ALPHA * h + (1.0 - ALPHA) * hn
    hout_ref[...] = hnew
    ehout_ref[...] = jnp.exp(hnew)


_row_spec = pl.BlockSpec((RB, H), lambda i: (i, 0))
_w_spec = pl.BlockSpec((H, H), lambda i: (0, 0))
_v_spec = pl.BlockSpec((1, H), lambda i: (0, 0))
_out2 = (jax.ShapeDtypeStruct((N, H), jnp.float32),
         jax.ShapeDtypeStruct((N, H), jnp.float32))

_proj = pl.pallas_call(
    _proj_body,
    grid=(_GRID,),
    in_specs=[_row_spec, _w_spec, _v_spec],
    out_specs=(_row_spec, _row_spec),
    out_shape=_out2,
)

_dense = pl.pallas_call(
    _dense_body,
    grid=(_GRID,),
    in_specs=[_row_spec, _row_spec, _row_spec, _w_spec, _w_spec, _v_spec,
              _v_spec, _v_spec],
    out_specs=(_row_spec, _row_spec),
    out_shape=_out2,
)


def kernel(x, edge_src, edge_dst, W_in, b_in, LW, Lb, Lg, Lbe):
    src = edge_src.astype(jnp.int32)
    dst = edge_dst.astype(jnp.int32)

    h, eh = _proj(x, W_in, b_in.reshape(1, H))
    ztile = jnp.zeros((ZROWS, H), jnp.float32)

    for i in range(L):
        s_full = _segsum(eh, src, dst, ztile)
        h, eh = _dense(h, s_full[0, :N], s_full[1, :N], LW[i, :H], LW[i, H:],
                       Lb[i].reshape(1, H), Lg[i].reshape(1, H),
                       Lbe[i].reshape(1, H))
    return h


# speedup vs baseline: 1.0425x; 1.0425x over previous
import jax
import jax.numpy as jnp
from jax import lax
from jax.experimental import pallas as pl
from jax.experimental.pallas import tpu as pltpu
from jax.experimental.pallas import tpu_sc as plsc

N = 10000
E = 320000
D = 128
H = 128
L = 3
EPS = 1e-30
ALPHA = 0.5

NC = 2
NS = 16
NW = NC * NS
NOUT = 10112
CHUNK = 128
EPT = E // NW
NFC = EPT // CHUNK
REM = EPT - NFC * CHUNK
ZROWS = NOUT // NS
ZR = 32

_mesh = plsc.VectorSubcoreMesh(core_axis_name="c", subcore_axis_name="s")


def _segsum_body(eh_hbm, src_hbm, dst_hbm, z_hbm, out_hbm,
                 table, sidx3, dstA, dstB, dstC, rowsA, rowsB, rowsC,
                 dstt, gsA, gsB, gsC, ssA, ssB, ssC):
    cid = lax.axis_index("c")
    sid = lax.axis_index("s")
    wid = sid * NC + cid
    ebase = wid * EPT
    zbase = sid * ZROWS

    pltpu.sync_copy(z_hbm, table.at[pl.ds(zbase, ZROWS)])

    plsc.subcore_barrier()

    def _triple(i, _):
        off = ebase + (i * 3) * CHUNK
        pltpu.sync_copy(src_hbm.at[pl.ds(off, 3 * CHUNK)], sidx3)
        pltpu.sync_copy(dst_hbm.at[pl.ds(off, CHUNK)], dstA)
        dA = pltpu.async_copy(eh_hbm.at[sidx3.at[pl.ds(0, CHUNK)]],
                              rowsA, gsA)
        pltpu.sync_copy(dst_hbm.at[pl.ds(off + CHUNK, CHUNK)], dstB)
        dB = pltpu.async_copy(eh_hbm.at[sidx3.at[pl.ds(CHUNK, CHUNK)]],
                              rowsB, gsB)
        pltpu.sync_copy(dst_hbm.at[pl.ds(off + 2 * CHUNK, CHUNK)], dstC)
        dC = pltpu.async_copy(eh_hbm.at[sidx3.at[pl.ds(2 * CHUNK, CHUNK)]],
                              rowsC, gsC)
        dA.wait()
        sA = pltpu.async_copy(rowsA, table.at[dstA], ssA, add=True)
        dB.wait()
        sB = pltpu.async_copy(rowsB, table.at[dstB], ssB, add=True)
        dC.wait()
        sC = pltpu.async_copy(rowsC, table.at[dstC], ssC, add=True)
        sA.wait()
        sB.wait()
        sC.wait()
        return 0

    lax.fori_loop(0, NFC // 3, _triple, 0)

    toff = ebase + NFC * CHUNK
    pltpu.sync_copy(src_hbm.at[pl.ds(toff, REM)], sidx3.at[pl.ds(0, REM)])
    pltpu.sync_copy(dst_hbm.at[pl.ds(toff, REM)], dstt)
    pltpu.async_copy(eh_hbm.at[sidx3.at[pl.ds(0, REM)]],
                     rowsA.at[pl.ds(0, REM)], gsA).wait()
    pltpu.sync_copy(rowsA.at[pl.ds(0, REM)], table.at[dstt], add=True)

    plsc.subcore_barrier()

    pltpu.sync_copy(table.at[pl.ds(zbase, ZROWS)],
                    out_hbm.at[cid, pl.ds(zbase, ZROWS)])


_segsum = pl.kernel(
    _segsum_body,
    out_type=jax.ShapeDtypeStruct((NC, NOUT, H), jnp.float32),
    mesh=_mesh,
    scratch_types=[
        pltpu.VMEM_SHARED((NOUT, H), jnp.float32),
        pltpu.VMEM((3 * CHUNK,), jnp.int32),
        pltpu.VMEM((CHUNK,), jnp.int32),
        pltpu.VMEM((CHUNK,), jnp.int32),
        pltpu.VMEM((CHUNK,), jnp.int32),
        pltpu.VMEM((CHUNK, H), jnp.float32),
        pltpu.VMEM((CHUNK, H), jnp.float32),
        pltpu.VMEM((CHUNK, H), jnp.float32),
        pltpu.VMEM((REM,), jnp.int32),
        pltpu.SemaphoreType.DMA,
        pltpu.SemaphoreType.DMA,
        pltpu.SemaphoreType.DMA,
        pltpu.SemaphoreType.DMA,
        pltpu.SemaphoreType.DMA,
        pltpu.SemaphoreType.DMA,
    ],
)


RB = 1000
_GRID = N // RB


def _proj_body(x_ref, w_ref, b_ref, h_ref, eh_ref):
    h = lax.dot_general(x_ref[...], w_ref[...], (((1,), (0,)), ((), ())),
                        precision=lax.Precision.HIGHEST,
                        preferred_element_type=jnp.float32) + b_ref[...]
    h_ref[...] = h
    eh_ref[...] = jnp.exp(h)


def _dense_body(h_ref, s0_ref, s1_ref, wt_ref, wb_ref, b_ref, g_ref, be_ref,
                hout_ref, ehout_ref):
    h = h_ref[...]
    s = s0_ref[...] + s1_ref[...]
    agg = jnp.where(s > 0, jnp.log(jnp.maximum(s, EPS)), 0.0)
    hn = (lax.dot_general(h, wt_ref[...], (((1,), (0,)), ((), ())),
                          precision=lax.Precision.HIGHEST,
                          preferred_element_type=jnp.float32)
          + lax.dot_general(agg, wb_ref[...], (((1,), (0,)), ((), ())),
                            precision=lax.Precision.HIGHEST,
                            preferred_element_type=jnp.float32)
          + b_ref[...])
    mu = jnp.mean(hn, axis=1, keepdims=True)
    var = jnp.mean((hn - mu) ** 2, axis=1, keepdims=True)
    hn = (hn - mu) / jnp.sqrt(var + 1e-5) * g_ref[...] + be_ref[...]
    hn = jnp.maximum(hn, 0.0)
    hnew = ALPHA * h + (1.0 - ALPHA) * hn
    hout_ref[...] = hnew
    ehout_ref[...] = jnp.exp(hnew)


_row_spec = pl.BlockSpec((RB, H), lambda i: (i, 0))
_w_spec = pl.BlockSpec((H, H), lambda i: (0, 0))
_v_spec = pl.BlockSpec((1, H), lambda i: (0, 0))
_out2 = (jax.ShapeDtypeStruct((N, H), jnp.float32),
         jax.ShapeDtypeStruct((N, H), jnp.float32))

_proj = pl.pallas_call(
    _proj_body,
    grid=(_GRID,),
    in_specs=[_row_spec, _w_spec, _v_spec],
    out_specs=(_row_spec, _row_spec),
    out_shape=_out2,
)

_dense = pl.pallas_call(
    _dense_body,
    grid=(_GRID,),
    in_specs=[_row_spec, _row_spec, _row_spec, _w_spec, _w_spec, _v_spec,
              _v_spec, _v_spec],
    out_specs=(_row_spec, _row_spec),
    out_shape=_out2,
)


def kernel(x, edge_src, edge_dst, W_in, b_in, LW, Lb, Lg, Lbe):
    src = edge_src.astype(jnp.int32)
    dst = edge_dst.astype(jnp.int32)

    h, eh = _proj(x, W_in, b_in.reshape(1, H))
    ztile = jnp.zeros((ZROWS, H), jnp.float32)

    for i in range(L):
        s_full = _segsum(eh, src, dst, ztile)
        h, eh = _dense(h, s_full[0, :N], s_full[1, :N], LW[i, :H], LW[i, H:],
                       Lb[i].reshape(1, H), Lg[i].reshape(1, H),
                       Lbe[i].reshape(1, H))
    return h
